# baseline (speedup 1.0000x reference)
import functools

import jax
import jax.numpy as jnp
from jax import lax
from jax.experimental import pallas as pl
from jax.experimental.pallas import tpu as pltpu
from jax.experimental.pallas import tpu_sc as plsc

R = 128
N = 32768
L = 16
NC = 2
NS = 16
NW = NC * NS
SC_RPW = 2
SC_ROWS = NW * SC_RPW
ACCS = 16
STEPS = N // (ACCS * L)

TC_BLOCK = 8
TC_CHUNK = 1024
_INT_MAX = 2**31 - 1


_SC_SCRATCH = [
    pltpu.VMEM((2, N), jnp.float32),
    pltpu.VMEM((L,), jnp.int32),
    pltpu.VMEM((L,), jnp.float32),
    pltpu.VMEM((L,), jnp.int32),
    pltpu.SemaphoreType.DMA,
    pltpu.SemaphoreType.DMA,
]


def _argmin_sc_body(x_hbm, out_hbm, buf, outbuf, redv, redi, sem0, sem1):
    cid = lax.axis_index("c")
    sid = lax.axis_index("s")
    wid = cid * NS + sid
    base_iota = lax.iota(jnp.int32, L)
    row0 = wid * SC_RPW

    pltpu.make_async_copy(x_hbm.at[row0], buf.at[0], sem0).start()
    pltpu.make_async_copy(x_hbm.at[row0 + 1], buf.at[1], sem1).start()

    def scan_row(slot):
        mv0 = tuple(
            jnp.full((L,), jnp.inf, dtype=jnp.float32) for _ in range(ACCS)
        )
        mt0 = tuple(jnp.zeros((L,), dtype=jnp.int32) for _ in range(ACCS))

        @plsc.parallel_loop(0, STEPS, 1, unroll=2, carry=(mv0, mt0))
        def _scan(t, carry):
            mvs, mts = carry
            tb = jnp.full((L,), t, dtype=jnp.int32)
            new_mvs = []
            new_mts = []
            for k in range(ACCS):
                v = buf[slot, pl.ds(t * (ACCS * L) + k * L, L)]
                m = v < mvs[k]
                new_mvs.append(jnp.where(m, v, mvs[k]))
                new_mts.append(jnp.where(m, tb, mts[k]))
            return tuple(new_mvs), tuple(new_mts)

        mvs, mts = _scan
        mv = mvs[0]
        mi = mts[0] * (ACCS * L) + base_iota
        for k in range(1, ACCS):
            fi = mts[k] * (ACCS * L) + (k * L + base_iota)
            take = (mvs[k] < mv) | ((mvs[k] == mv) & (fi < mi))
            mv = jnp.where(take, mvs[k], mv)
            mi = jnp.where(take, fi, mi)

        for sh in (8, 4, 2, 1):
            redv[...] = mv
            redi[...] = mi
            perm = base_iota ^ sh
            ov = plsc.load_gather(redv, [perm])
            oi = plsc.load_gather(redi, [perm])
            take = (ov < mv) | ((ov == mv) & (oi < mi))
            mv = jnp.where(take, ov, mv)
            mi = jnp.where(take, oi, mi)
        return mi

    pltpu.make_async_copy(x_hbm.at[row0], buf.at[0], sem0).wait()
    mi_a = scan_row(0)
    pltpu.make_async_copy(x_hbm.at[row0 + 1], buf.at[1], sem1).wait()
    mi_b = scan_row(1)

    outbuf[...] = jnp.where(base_iota == 0, mi_a, mi_b)
    pltpu.sync_copy(outbuf, out_hbm.at[wid])


_argmin_sc = functools.partial(
    pl.kernel,
    mesh=plsc.VectorSubcoreMesh(core_axis_name="c", subcore_axis_name="s"),
    out_type=jax.ShapeDtypeStruct((NW, L), jnp.int32),
    scratch_types=_SC_SCRATCH,
    compiler_params=pltpu.CompilerParams(
        needs_layout_passes=False, skip_device_barrier=True
    ),
)(_argmin_sc_body)


def _tc_body(x_ref, o_ref):
    bmv = jnp.full((TC_BLOCK, TC_CHUNK), jnp.inf, dtype=jnp.float32)
    bci = jnp.zeros((TC_BLOCK, TC_CHUNK), dtype=jnp.int32)
    for c in range(N // TC_CHUNK):
        v = x_ref[:, pl.ds(c * TC_CHUNK, TC_CHUNK)]
        m = v < bmv
        bmv = jnp.where(m, v, bmv)
        bci = jnp.where(m, jnp.int32(c), bci)

    rowmin = jnp.min(bmv, axis=1, keepdims=True)
    pos = lax.broadcasted_iota(jnp.int32, (TC_BLOCK, TC_CHUNK), 1)
    flat = bci * TC_CHUNK + pos
    cand = jnp.where(bmv == rowmin, flat, _INT_MAX)
    o_ref[...] = jnp.min(cand, axis=1, keepdims=True)


_argmin_tc = pl.pallas_call(
    _tc_body,
    grid=((R - SC_ROWS) // TC_BLOCK,),
    in_specs=[
        pl.BlockSpec((TC_BLOCK, N), lambda i: (i + SC_ROWS // TC_BLOCK, 0))
    ],
    out_specs=pl.BlockSpec((TC_BLOCK, 1), lambda i: (i + SC_ROWS // TC_BLOCK, 0)),
    out_shape=jax.ShapeDtypeStruct((R, 1), jnp.int32),
)


def kernel(x):
    tc_full = _argmin_tc(x)
    sc_part = _argmin_sc(x)
    sc_rows = sc_part[:, :SC_RPW].reshape(SC_ROWS, 1)
    return lax.dynamic_update_slice(tc_full, sc_rows, (0, 0))


# speedup vs baseline: 1.1012x; 1.1012x over previous
import functools

import jax
import jax.numpy as jnp
from jax import lax
from jax.experimental import pallas as pl
from jax.experimental.pallas import tpu as pltpu
from jax.experimental.pallas import tpu_sc as plsc

R = 128
N = 32768
L = 16
NC = 2
NS = 16
NW = NC * NS
SC_RPW = 2
SC_ROWS = NW * SC_RPW
ACCS = 8
STEPS = N // (ACCS * L)

TC_BLOCK = 8
TC_CHUNK = 1024
_INT_MAX = 2**31 - 1


_SC_SCRATCH = [
    pltpu.VMEM((2, N), jnp.float32),
    pltpu.VMEM((L,), jnp.int32),
    pltpu.VMEM((L,), jnp.float32),
    pltpu.VMEM((L,), jnp.int32),
    pltpu.SemaphoreType.DMA,
    pltpu.SemaphoreType.DMA,
]


def _argmin_sc_body(x_hbm, out_hbm, buf, outbuf, redv, redi, sem0, sem1):
    cid = lax.axis_index("c")
    sid = lax.axis_index("s")
    wid = cid * NS + sid
    base_iota = lax.iota(jnp.int32, L)
    row0 = wid * SC_RPW

    pltpu.make_async_copy(x_hbm.at[row0], buf.at[0], sem0).start()
    pltpu.make_async_copy(x_hbm.at[row0 + 1], buf.at[1], sem1).start()

    def scan_row(slot):
        mv0 = tuple(
            jnp.full((L,), jnp.inf, dtype=jnp.float32) for _ in range(ACCS)
        )
        mt0 = tuple(jnp.zeros((L,), dtype=jnp.int32) for _ in range(ACCS))

        @plsc.parallel_loop(0, STEPS, 1, unroll=4, carry=(mv0, mt0))
        def _scan(t, carry):
            mvs, mts = carry
            tb = jnp.full((L,), t, dtype=jnp.int32)
            new_mvs = []
            new_mts = []
            for k in range(ACCS):
                v = buf[slot, pl.ds(t * (ACCS * L) + k * L, L)]
                m = v < mvs[k]
                new_mvs.append(jnp.where(m, v, mvs[k]))
                new_mts.append(jnp.where(m, tb, mts[k]))
            return tuple(new_mvs), tuple(new_mts)

        mvs, mts = _scan
        mv = mvs[0]
        mi = mts[0] * (ACCS * L) + base_iota
        for k in range(1, ACCS):
            fi = mts[k] * (ACCS * L) + (k * L + base_iota)
            take = (mvs[k] < mv) | ((mvs[k] == mv) & (fi < mi))
            mv = jnp.where(take, mvs[k], mv)
            mi = jnp.where(take, fi, mi)

        for sh in (8, 4, 2, 1):
            redv[...] = mv
            redi[...] = mi
            perm = base_iota ^ sh
            ov = plsc.load_gather(redv, [perm])
            oi = plsc.load_gather(redi, [perm])
            take = (ov < mv) | ((ov == mv) & (oi < mi))
            mv = jnp.where(take, ov, mv)
            mi = jnp.where(take, oi, mi)
        return mi

    pltpu.make_async_copy(x_hbm.at[row0], buf.at[0], sem0).wait()
    mi_a = scan_row(0)
    pltpu.make_async_copy(x_hbm.at[row0 + 1], buf.at[1], sem1).wait()
    mi_b = scan_row(1)

    outbuf[...] = jnp.where(base_iota == 0, mi_a, mi_b)
    pltpu.sync_copy(outbuf, out_hbm.at[wid])


_argmin_sc = functools.partial(
    pl.kernel,
    mesh=plsc.VectorSubcoreMesh(core_axis_name="c", subcore_axis_name="s"),
    out_type=jax.ShapeDtypeStruct((NW, L), jnp.int32),
    scratch_types=_SC_SCRATCH,
    compiler_params=pltpu.CompilerParams(
        needs_layout_passes=False, skip_device_barrier=True
    ),
)(_argmin_sc_body)


def _tc_body(x_ref, o_ref):
    bmv = jnp.full((TC_BLOCK, TC_CHUNK), jnp.inf, dtype=jnp.float32)
    bci = jnp.zeros((TC_BLOCK, TC_CHUNK), dtype=jnp.int32)
    for c in range(N // TC_CHUNK):
        v = x_ref[:, pl.ds(c * TC_CHUNK, TC_CHUNK)]
        m = v < bmv
        bmv = jnp.where(m, v, bmv)
        bci = jnp.where(m, jnp.int32(c), bci)

    rowmin = jnp.min(bmv, axis=1, keepdims=True)
    pos = lax.broadcasted_iota(jnp.int32, (TC_BLOCK, TC_CHUNK), 1)
    flat = bci * TC_CHUNK + pos
    cand = jnp.where(bmv == rowmin, flat, _INT_MAX)
    o_ref[...] = jnp.min(cand, axis=1, keepdims=True)


_argmin_tc = pl.pallas_call(
    _tc_body,
    grid=((R - SC_ROWS) // TC_BLOCK,),
    in_specs=[
        pl.BlockSpec((TC_BLOCK, N), lambda i: (i + SC_ROWS // TC_BLOCK, 0))
    ],
    out_specs=pl.BlockSpec((TC_BLOCK, 1), lambda i: (i + SC_ROWS // TC_BLOCK, 0)),
    out_shape=jax.ShapeDtypeStruct((R, 1), jnp.int32),
)


def kernel(x):
    tc_full = _argmin_tc(x)
    sc_part = _argmin_sc(x)
    sc_rows = sc_part[:, :SC_RPW].reshape(SC_ROWS, 1)
    return lax.dynamic_update_slice(tc_full, sc_rows, (0, 0))
